# baseline (speedup 1.0000x reference)
import functools

import jax
import jax.numpy as jnp
from jax import lax
from jax.experimental import pallas as pl
from jax.experimental.pallas import tpu as pltpu
from jax.experimental.pallas import tpu_sc as plsc

NUM_COUNTIES = 100000
BATCH = 16384
NC = 2
NS = 16
NW = NC * NS
ROWS_W = BATCH // NW
IDX_CHUNK = 128
N_CHUNKS = ROWS_W // IDX_CHUNK


def _softplus(x):
  t = jnp.exp(-jnp.abs(x))
  s = t / (t + 2.0)
  s2 = s * s
  p = 1.0 + s2 * (1.0 / 3.0 + s2 * (0.2 + s2 * (1.0 / 7.0 + s2 * (1.0 / 9.0))))
  return jnp.maximum(x, 0.0) + 2.0 * s * p


def _body(d0_h, d1_h, d2_h, county_h, raw_h, uloc_h, uscale_h, epsu_h,
          out_h,
          county_v, eidx_v, loc_v, scale_v, eps_v, d0_v, d1_v, d2_v, raw_v,
          out_v, sem):
  wid = lax.axis_index("s") * NC + lax.axis_index("c")
  base = wid * ROWS_W

  copies = [
      pltpu.async_copy(d0_h.at[pl.ds(base, ROWS_W)], d0_v, sem),
      pltpu.async_copy(d1_h.at[pl.ds(base, ROWS_W)], d1_v, sem),
      pltpu.async_copy(d2_h.at[pl.ds(base, ROWS_W)], d2_v, sem),
      pltpu.async_copy(raw_h, raw_v, sem),
  ]
  pltpu.sync_copy(county_h.at[pl.ds(base, ROWS_W)], county_v)

  iota = lax.iota(jnp.int32, 16)
  half = lax.shift_right_logical(iota, 1)
  col = lax.bitwise_and(iota, 1)
  in_bounds = lax.GatherScatterMode.PROMISE_IN_BOUNDS

  def idx_chunk(s, carry):
    cv = county_v[pl.ds(16 * s, 16)]
    for h in range(2):
      ce = jnp.take_along_axis(cv, 8 * h + half, axis=0, mode=in_bounds)
      eidx_v[pl.ds(32 * s + 16 * h, 16)] = 2 * ce + col
    return carry

  lax.fori_loop(0, ROWS_W // 16, idx_chunk, 0, unroll=2)

  for tbl_h, tbl_v in ((uloc_h, loc_v), (uscale_h, scale_v), (epsu_h, eps_v)):
    for j in range(2 * ROWS_W // IDX_CHUNK):
      copies.append(pltpu.async_copy(
          tbl_h.at[eidx_v.at[pl.ds(j * IDX_CHUNK, IDX_CHUNK)]],
          tbl_v.at[pl.ds(j * IDX_CHUNK, IDX_CHUNK)], sem))
  for c in copies:
    c.wait()

  def coef(r):
    blt = raw_v[pl.ds(r * 16, 16)]
    bst = raw_v[pl.ds((3 + r) * 16, 16)]
    ebt = raw_v[pl.ds((6 + r) * 16, 16)]
    return blt + ebt * _softplus(bst)

  coef0 = coef(0)
  coef1 = coef(1)
  coef2 = coef(2)
  icpt = raw_v[pl.ds(144, 16)]

  def pair(s, carry):
    dv0 = d0_v[pl.ds(16 * s, 16)]
    dv1 = d1_v[pl.ds(16 * s, 16)]
    dv2 = d2_v[pl.ds(16 * s, 16)]
    for h in range(2):
      t = 2 * s + h
      idx = 8 * h + half
      e0 = jnp.take_along_axis(dv0, idx, axis=0, mode=in_bounds)
      e1 = jnp.take_along_axis(dv1, idx, axis=0, mode=in_bounds)
      e2 = jnp.take_along_axis(dv2, idx, axis=0, mode=in_bounds)
      lc = loc_v[pl.ds(16 * t, 16)]
      sc = scale_v[pl.ds(16 * t, 16)]
      ec = eps_v[pl.ds(16 * t, 16)]
      res = icpt + e0 * coef0 + e1 * coef1 + e2 * coef2 + lc + ec * _softplus(sc)
      out_v[pl.ds(16 * t, 16)] = res
    return carry

  lax.fori_loop(0, ROWS_W // 16, pair, 0, unroll=2)

  pltpu.sync_copy(out_v, out_h.at[pl.ds(base * 2, ROWS_W * 2)])


@jax.jit
def _run(d0, d1, d2, county, raw, u_loc, u_scale, eps_u):
  mesh = plsc.VectorSubcoreMesh(
      core_axis_name="c", subcore_axis_name="s", num_cores=NC, num_subcores=NS)
  f = pl.kernel(
      _body,
      out_type=jax.ShapeDtypeStruct((BATCH * 2,), jnp.float32),
      mesh=mesh,
      scratch_types=[
          pltpu.VMEM((ROWS_W,), jnp.int32),
          pltpu.VMEM((ROWS_W * 2,), jnp.int32),
          pltpu.VMEM((ROWS_W * 2,), jnp.float32),
          pltpu.VMEM((ROWS_W * 2,), jnp.float32),
          pltpu.VMEM((ROWS_W * 2,), jnp.float32),
          pltpu.VMEM((ROWS_W,), jnp.float32),
          pltpu.VMEM((ROWS_W,), jnp.float32),
          pltpu.VMEM((ROWS_W,), jnp.float32),
          pltpu.VMEM((160,), jnp.float32),
          pltpu.VMEM((ROWS_W * 2,), jnp.float32),
          pltpu.SemaphoreType.DMA,
      ],
  )
  return f(d0, d1, d2, county, raw, u_loc, u_scale, eps_u)


def kernel(d0, d1, d2, county, beta_loc, beta_scale, u_loc, u_scale,
           intercept, eps_beta, eps_u):
  raw = jnp.concatenate([
      jnp.tile(beta_loc, (1, 8)).reshape(-1),
      jnp.tile(beta_scale, (1, 8)).reshape(-1),
      jnp.tile(eps_beta, (1, 8)).reshape(-1),
      jnp.tile(intercept, 8),
  ])
  out = _run(d0, d1, d2, county, raw,
             u_loc.reshape(-1), u_scale.reshape(-1), eps_u.reshape(-1))
  return out.reshape(BATCH, 2)
